# baseline (speedup 1.0000x reference)
import functools

import jax
import jax.numpy as jnp
from jax import lax
from jax.experimental import pallas as pl
from jax.experimental.pallas import tpu as pltpu, tpu_sc as plsc

_BATCH = 4
_SEQ = 8192
_D = 64
_MAXLEN = 10000

_NC = 2
_NS = 16
_NW = _NC * _NS

_WIDTH = 512
_K = _WIDTH // _D
_PROWS = _BATCH * _SEQ // _K
_TAB_PROWS = _MAXLEN * _D // _WIDTH
_SEQ_PROWS = _SEQ // _K

_PR_PER_W = _PROWS // _NW
_CHUNK = 64
_NCH = _PR_PER_W // _CHUNK
_WORKERS_PER_BATCH = _SEQ_PROWS // _PR_PER_W
_VPR = _WIDTH // 16


def _pos_add_body(x_hbm, pos_hbm, tab_hbm, out_hbm, bufx, buft, idx_v, sem):
    wid = lax.axis_index("s") * _NC + lax.axis_index("c")
    row_base = wid * _PR_PER_W
    l_base = (wid % _WORKERS_PER_BATCH) * _PR_PER_W

    for c in range(_NCH):
        r0 = row_base + c * _CHUNK
        l0 = l_base + c * _CHUNK
        pltpu.sync_copy(pos_hbm.at[pl.ds(l0, _CHUNK)], idx_v)
        pltpu.sync_copy(x_hbm.at[pl.ds(r0, _CHUNK)], bufx)
        pltpu.async_copy(tab_hbm.at[idx_v], buft, sem).wait()

        def add_body(i, _):
            r = i // _VPR
            s = pl.ds((i % _VPR) * 16, 16)
            bufx[r, s] = bufx[r, s] + buft[r, s]
            return 0

        lax.fori_loop(0, _CHUNK * _VPR, add_body, 0)
        pltpu.sync_copy(bufx, out_hbm.at[pl.ds(r0, _CHUNK)])


def _make_pos_add(interpret=False):
    return functools.partial(
        pl.kernel,
        out_type=jax.ShapeDtypeStruct((_PROWS, _WIDTH), jnp.float32),
        mesh=plsc.VectorSubcoreMesh(core_axis_name="c", subcore_axis_name="s"),
        scratch_types=[
            pltpu.VMEM((_CHUNK, _WIDTH), jnp.float32),
            pltpu.VMEM((_CHUNK, _WIDTH), jnp.float32),
            pltpu.VMEM((_CHUNK,), jnp.int32),
            pltpu.SemaphoreType.DMA,
        ],
        interpret=interpret,
    )(_pos_add_body)


_pos_add = _make_pos_add()


def kernel(x, pos_emb_table):
    positions = jnp.arange(_SEQ_PROWS, dtype=jnp.int32)
    x2d = x.reshape(_PROWS, _WIDTH)
    tab2d = pos_emb_table.reshape(_TAB_PROWS, _WIDTH)
    out2d = _pos_add(x2d, positions, tab2d)
    return out2d.reshape(_BATCH, _SEQ, _D)


# speedup vs baseline: 1.1861x; 1.1861x over previous
import functools

import jax
import jax.numpy as jnp
from jax import lax
from jax.experimental import pallas as pl
from jax.experimental.pallas import tpu as pltpu, tpu_sc as plsc

_BATCH = 4
_SEQ = 8192
_D = 64
_MAXLEN = 10000

_NC = 2
_NS = 16
_NW = _NC * _NS

_WIDTH = 512
_K = _WIDTH // _D
_PROWS = _BATCH * _SEQ // _K
_TAB_PROWS = _MAXLEN * _D // _WIDTH
_SEQ_PROWS = _SEQ // _K

_PR_PER_W = _PROWS // _NW
_CHUNK = 64
_NCH = _PR_PER_W // _CHUNK
_WORKERS_PER_BATCH = _SEQ_PROWS // _PR_PER_W
_VPR = _WIDTH // 16


def _pos_add_body(x_hbm, pos_hbm, tab_hbm, out_hbm, bufx, buft, idx_v, semx, semt):
    wid = lax.axis_index("s") * _NC + lax.axis_index("c")
    row_base = wid * _PR_PER_W
    l_base = (wid % _WORKERS_PER_BATCH) * _PR_PER_W

    for c in range(_NCH):
        r0 = row_base + c * _CHUNK
        l0 = l_base + c * _CHUNK
        pltpu.sync_copy(pos_hbm.at[pl.ds(l0, _CHUNK)], idx_v)
        cx = pltpu.async_copy(x_hbm.at[pl.ds(r0, _CHUNK)], bufx, semx)
        ct = pltpu.async_copy(tab_hbm.at[idx_v], buft, semt)
        cx.wait()
        ct.wait()

        @plsc.parallel_loop(0, _CHUNK * _VPR, unroll=8)
        def _add(i):
            r = i // _VPR
            s = pl.ds((i % _VPR) * 16, 16)
            plsc.addupdate(bufx.at[r, s], buft[r, s])

        pltpu.sync_copy(bufx, out_hbm.at[pl.ds(r0, _CHUNK)])


def _make_pos_add(interpret=False):
    return functools.partial(
        pl.kernel,
        out_type=jax.ShapeDtypeStruct((_PROWS, _WIDTH), jnp.float32),
        mesh=plsc.VectorSubcoreMesh(core_axis_name="c", subcore_axis_name="s"),
        scratch_types=[
            pltpu.VMEM((_CHUNK, _WIDTH), jnp.float32),
            pltpu.VMEM((_CHUNK, _WIDTH), jnp.float32),
            pltpu.VMEM((_CHUNK,), jnp.int32),
            pltpu.SemaphoreType.DMA,
            pltpu.SemaphoreType.DMA,
        ],
        interpret=interpret,
    )(_pos_add_body)


_pos_add = _make_pos_add()


def kernel(x, pos_emb_table):
    positions = jnp.arange(_SEQ_PROWS, dtype=jnp.int32)
    x2d = x.reshape(_PROWS, _WIDTH)
    tab2d = pos_emb_table.reshape(_TAB_PROWS, _WIDTH)
    out2d = _pos_add(x2d, positions, tab2d)
    return out2d.reshape(_BATCH, _SEQ, _D)


# speedup vs baseline: 1.5866x; 1.3376x over previous
import functools

import jax
import jax.numpy as jnp
from jax import lax
from jax.experimental import pallas as pl
from jax.experimental.pallas import tpu as pltpu, tpu_sc as plsc

_BATCH = 4
_SEQ = 8192
_D = 64

_NC = 2
_NS = 16
_NW = _NC * _NS

_LPW = _SEQ // _NW
_VPR = _D // 16
_NV = _LPW * _VPR


def _pos_add_body(x_hbm, tab_hbm, out_hbm, bufx0, bufx1, buft,
                  semt, semx0, semx1, semo0, semo1):
    wid = lax.axis_index("s") * _NC + lax.axis_index("c")
    l0 = wid * _LPW

    bufs = (bufx0, bufx1)
    semx = (semx0, semx1)
    semo = (semo0, semo1)

    ct = pltpu.async_copy(tab_hbm.at[pl.ds(l0, _LPW)], buft, semt)
    pltpu.async_copy(x_hbm.at[0, pl.ds(l0, _LPW)], bufx0, semx0)
    ct.wait()

    for b in range(_BATCH):
        cur = bufs[b % 2]
        pltpu.make_async_copy(x_hbm.at[b, pl.ds(l0, _LPW)], cur,
                              semx[b % 2]).wait()
        if b + 1 < _BATCH:
            nxt = bufs[(b + 1) % 2]
            if b >= 1:
                pltpu.make_async_copy(nxt, out_hbm.at[b - 1, pl.ds(l0, _LPW)],
                                      semo[(b + 1) % 2]).wait()
            pltpu.async_copy(x_hbm.at[b + 1, pl.ds(l0, _LPW)], nxt,
                             semx[(b + 1) % 2])

        @plsc.parallel_loop(0, _NV, unroll=8)
        def _add(i):
            r = i // _VPR
            s = pl.ds((i % _VPR) * 16, 16)
            plsc.addupdate(cur.at[r, s], buft[r, s])

        pltpu.async_copy(cur, out_hbm.at[b, pl.ds(l0, _LPW)], semo[b % 2])

    pltpu.make_async_copy(bufs[(_BATCH - 2) % 2],
                          out_hbm.at[_BATCH - 2, pl.ds(l0, _LPW)],
                          semo[(_BATCH - 2) % 2]).wait()
    pltpu.make_async_copy(bufs[(_BATCH - 1) % 2],
                          out_hbm.at[_BATCH - 1, pl.ds(l0, _LPW)],
                          semo[(_BATCH - 1) % 2]).wait()


def _make_pos_add(interpret=False):
    return functools.partial(
        pl.kernel,
        out_type=jax.ShapeDtypeStruct((_BATCH, _SEQ, _D), jnp.float32),
        mesh=plsc.VectorSubcoreMesh(core_axis_name="c", subcore_axis_name="s"),
        scratch_types=[
            pltpu.VMEM((_LPW, _D), jnp.float32),
            pltpu.VMEM((_LPW, _D), jnp.float32),
            pltpu.VMEM((_LPW, _D), jnp.float32),
            pltpu.SemaphoreType.DMA,
            pltpu.SemaphoreType.DMA,
            pltpu.SemaphoreType.DMA,
            pltpu.SemaphoreType.DMA,
            pltpu.SemaphoreType.DMA,
        ],
        interpret=interpret,
    )(_pos_add_body)


_pos_add = _make_pos_add()


def kernel(x, pos_emb_table):
    return _pos_add(x, pos_emb_table)


# speedup vs baseline: 1.5885x; 1.0012x over previous
import functools

import jax
import jax.numpy as jnp
from jax import lax
from jax.experimental import pallas as pl
from jax.experimental.pallas import tpu as pltpu, tpu_sc as plsc

_BATCH = 4
_SEQ = 8192
_D = 64

_NC = 2
_NS = 16
_NW = _NC * _NS

_LPW = _SEQ // _NW
_VPR = _D // 16
_NV = _LPW * _VPR


def _pos_add_body(x_hbm, tab_hbm, out_hbm, bufx0, bufx1, buft,
                  semt, semx0, semx1, semo0, semo1):
    wid = lax.axis_index("s") * _NC + lax.axis_index("c")
    l0 = wid * _LPW

    bufs = (bufx0, bufx1)
    semx = (semx0, semx1)
    semo = (semo0, semo1)

    ct = pltpu.async_copy(tab_hbm.at[pl.ds(l0, _LPW)], buft, semt)
    pltpu.async_copy(x_hbm.at[0, pl.ds(l0, _LPW)], bufx0, semx0)
    ct.wait()

    for b in range(_BATCH):
        cur = bufs[b % 2]
        pltpu.make_async_copy(x_hbm.at[b, pl.ds(l0, _LPW)], cur,
                              semx[b % 2]).wait()
        if b + 1 < _BATCH:
            nxt = bufs[(b + 1) % 2]
            if b >= 1:
                pltpu.make_async_copy(nxt, out_hbm.at[b - 1, pl.ds(l0, _LPW)],
                                      semo[(b + 1) % 2]).wait()
            pltpu.async_copy(x_hbm.at[b + 1, pl.ds(l0, _LPW)], nxt,
                             semx[(b + 1) % 2])

        @plsc.parallel_loop(0, _NV, unroll=8)
        def _add(i):
            r = i // _VPR
            s = pl.ds((i % _VPR) * 16, 16)
            plsc.addupdate(cur.at[r, s], buft[r, s])

        pltpu.async_copy(cur, out_hbm.at[b, pl.ds(l0, _LPW)], semo[b % 2])

    pltpu.make_async_copy(bufs[(_BATCH - 2) % 2],
                          out_hbm.at[_BATCH - 2, pl.ds(l0, _LPW)],
                          semo[(_BATCH - 2) % 2]).wait()
    pltpu.make_async_copy(bufs[(_BATCH - 1) % 2],
                          out_hbm.at[_BATCH - 1, pl.ds(l0, _LPW)],
                          semo[(_BATCH - 1) % 2]).wait()


def _make_pos_add(interpret=False):
    return functools.partial(
        pl.kernel,
        out_type=jax.ShapeDtypeStruct((_BATCH, _SEQ, _D), jnp.float32),
        mesh=plsc.VectorSubcoreMesh(core_axis_name="c", subcore_axis_name="s"),
        scratch_types=[
            pltpu.VMEM((_LPW, _D), jnp.float32),
            pltpu.VMEM((_LPW, _D), jnp.float32),
            pltpu.VMEM((_LPW, _D), jnp.float32),
            pltpu.SemaphoreType.DMA,
            pltpu.SemaphoreType.DMA,
            pltpu.SemaphoreType.DMA,
            pltpu.SemaphoreType.DMA,
            pltpu.SemaphoreType.DMA,
        ],
        compiler_params=pltpu.CompilerParams(use_tc_tiling_on_sc=True),
        interpret=interpret,
    )(_pos_add_body)


_pos_add = _make_pos_add()


def kernel(x, pos_emb_table):
    return _pos_add(x, pos_emb_table)


# speedup vs baseline: 3.4387x; 2.1647x over previous
import functools

import jax
import jax.numpy as jnp
from jax import lax
from jax.experimental import pallas as pl
from jax.experimental.pallas import tpu as pltpu, tpu_sc as plsc

_BATCH = 4
_SEQ = 8192
_D = 64

_NC = 2
_NS = 16
_NW = _NC * _NS

_NDC = 8
_DC = _D // _NDC
_NLQ = _NW // _NDC
_LQ = _SEQ // _NLQ
_NV = (_DC * _LQ) // 16


def _pos_add_body(x_hbm, tab_hbm, out_hbm, bufx0, bufx1, buft,
                  semt, semx0, semx1, semo0, semo1):
    wid = lax.axis_index("s") * _NC + lax.axis_index("c")
    dc0 = (wid // _NLQ) * _DC
    l0 = (wid % _NLQ) * _LQ

    bufs = (bufx0, bufx1)
    semx = (semx0, semx1)
    semo = (semo0, semo1)

    ct = pltpu.async_copy(tab_hbm.at[pl.ds(dc0, _DC), pl.ds(l0, _LQ)],
                          buft, semt)
    pltpu.async_copy(x_hbm.at[0, pl.ds(dc0, _DC), pl.ds(l0, _LQ)],
                     bufx0, semx0)
    ct.wait()

    for b in range(_BATCH):
        cur = bufs[b % 2]
        pltpu.make_async_copy(x_hbm.at[b, pl.ds(dc0, _DC), pl.ds(l0, _LQ)],
                              cur, semx[b % 2]).wait()
        if b + 1 < _BATCH:
            nxt = bufs[(b + 1) % 2]
            if b >= 1:
                pltpu.make_async_copy(
                    nxt, out_hbm.at[b - 1, pl.ds(dc0, _DC), pl.ds(l0, _LQ)],
                    semo[(b + 1) % 2]).wait()
            pltpu.async_copy(x_hbm.at[b + 1, pl.ds(dc0, _DC), pl.ds(l0, _LQ)],
                             nxt, semx[(b + 1) % 2])

        @plsc.parallel_loop(0, _NV, unroll=8)
        def _add(i):
            r = i // (_LQ // 16)
            s = pl.ds((i % (_LQ // 16)) * 16, 16)
            plsc.addupdate(cur.at[r, s], buft[r, s])

        pltpu.async_copy(cur, out_hbm.at[b, pl.ds(dc0, _DC), pl.ds(l0, _LQ)],
                         semo[b % 2])

    pltpu.make_async_copy(bufs[(_BATCH - 2) % 2],
                          out_hbm.at[_BATCH - 2, pl.ds(dc0, _DC), pl.ds(l0, _LQ)],
                          semo[(_BATCH - 2) % 2]).wait()
    pltpu.make_async_copy(bufs[(_BATCH - 1) % 2],
                          out_hbm.at[_BATCH - 1, pl.ds(dc0, _DC), pl.ds(l0, _LQ)],
                          semo[(_BATCH - 1) % 2]).wait()


def _make_pos_add(interpret=False):
    return functools.partial(
        pl.kernel,
        out_type=jax.ShapeDtypeStruct((_BATCH, _D, _SEQ), jnp.float32),
        mesh=plsc.VectorSubcoreMesh(core_axis_name="c", subcore_axis_name="s"),
        scratch_types=[
            pltpu.VMEM((_DC, _LQ), jnp.float32),
            pltpu.VMEM((_DC, _LQ), jnp.float32),
            pltpu.VMEM((_DC, _LQ), jnp.float32),
            pltpu.SemaphoreType.DMA,
            pltpu.SemaphoreType.DMA,
            pltpu.SemaphoreType.DMA,
            pltpu.SemaphoreType.DMA,
            pltpu.SemaphoreType.DMA,
        ],
        interpret=interpret,
    )(_pos_add_body)


_pos_add = _make_pos_add()


def kernel(x, pos_emb_table):
    xt = jnp.transpose(x, (0, 2, 1))
    tabt = jnp.transpose(pos_emb_table)
    outt = _pos_add(xt, tabt)
    return jnp.transpose(outt, (0, 2, 1))
